# baseline (speedup 1.0000x reference)
import functools

import jax
import jax.numpy as jnp
from jax import lax
from jax.experimental import pallas as pl
from jax.experimental.pallas import tpu as pltpu, tpu_sc as plsc

_NC, _NS = 2, 16
_NW = _NC * _NS
_CHUNK_ROWS = 32


def _make_sc_kernel(B, L, D):
    rows_per_w = L // _NW
    n_chunks = rows_per_w // _CHUNK_ROWS
    chunk = _CHUNK_ROWS * D
    n_vec = chunk // 16
    n_units = n_chunks * B
    mesh = plsc.VectorSubcoreMesh(core_axis_name="c", subcore_axis_name="s")

    @functools.partial(
        pl.kernel,
        out_type=jax.ShapeDtypeStruct((B * L * D,), jnp.float32),
        mesh=mesh,
        scratch_types=[
            pltpu.VMEM((2, chunk), jnp.float32),
            pltpu.VMEM((2, chunk), jnp.float32),
            pltpu.SemaphoreType.DMA,
            pltpu.SemaphoreType.DMA,
            pltpu.SemaphoreType.DMA,
            pltpu.SemaphoreType.DMA,
            pltpu.SemaphoreType.DMA,
            pltpu.SemaphoreType.DMA,
        ],
    )
    def sc_kernel(x_hbm, pos_hbm, out_hbm, pos_v, x_v,
                  in_s0, in_s1, out_s0, out_s1, pos_s0, pos_s1):
        in_s = (in_s0, in_s1)
        out_s = (out_s0, out_s1)
        pos_s = (pos_s0, pos_s1)
        w = lax.axis_index("s") * _NC + lax.axis_index("c")
        row0 = w * rows_per_w

        def x_off(u):
            ci, b = divmod(u, B)
            return b * L * D + (row0 + ci * _CHUNK_ROWS) * D

        def run_add(xbuf, pbuf):
            @plsc.parallel_loop(0, chunk, step=16, unroll=8)
            def add_body(i):
                pv = pos_v[pbuf, pl.ds(i, 16)]
                plsc.addupdate(x_v.at[xbuf, pl.ds(i, 16)], pv)

        hp = {0: pltpu.async_copy(
            pos_hbm.at[pl.ds(row0 * D, chunk)], pos_v.at[0], pos_s[0])}
        hx = {0: pltpu.async_copy(
            x_hbm.at[pl.ds(x_off(0), chunk)], x_v.at[0], in_s[0])}
        hout = {}
        for u in range(n_units):
            ci, b = divmod(u, B)
            pbuf = ci % 2
            xbuf = u % 2
            hx[u].wait()
            if b == 0:
                hp[ci].wait()
                if ci + 1 < n_chunks:
                    p_off = (row0 + (ci + 1) * _CHUNK_ROWS) * D
                    hp[ci + 1] = pltpu.async_copy(
                        pos_hbm.at[pl.ds(p_off, chunk)],
                        pos_v.at[(ci + 1) % 2], pos_s[(ci + 1) % 2])
            if u + 1 < n_units:
                if u - 1 >= 0:
                    hout[u - 1].wait()
                hx[u + 1] = pltpu.async_copy(
                    x_hbm.at[pl.ds(x_off(u + 1), chunk)],
                    x_v.at[(u + 1) % 2], in_s[(u + 1) % 2])
            run_add(xbuf, pbuf)
            hout[u] = pltpu.async_copy(
                x_v.at[xbuf], out_hbm.at[pl.ds(x_off(u), chunk)],
                out_s[xbuf])
        hout[n_units - 2].wait()
        hout[n_units - 1].wait()

    return sc_kernel


def kernel(x, pos_table):
    B, L, D = x.shape
    out_flat = _make_sc_kernel(B, L, D)(
        x.reshape(B * L * D), pos_table[:L].reshape(L * D)
    )
    return out_flat.reshape(B, L, D)


# speedup vs baseline: 5.1076x; 5.1076x over previous
import jax
import jax.numpy as jnp
from jax.experimental import pallas as pl


_BLK_L = 1024


def _add_kernel(x_ref, pos_ref, o_ref):
    o_ref[...] = x_ref[...] + pos_ref[...][None]


def kernel(x, pos_table):
    B, L, D = x.shape
    blk_l = 2048
    blk_b = 1
    grid = (L // blk_l, B // blk_b)
    return pl.pallas_call(
        _add_kernel,
        grid=grid,
        in_specs=[
            pl.BlockSpec((blk_b, blk_l, D), lambda l, b: (b, l, 0)),
            pl.BlockSpec((blk_l, D), lambda l, b: (l, 0)),
        ],
        out_specs=pl.BlockSpec((blk_b, blk_l, D), lambda l, b: (b, l, 0)),
        out_shape=jax.ShapeDtypeStruct((B, L, D), x.dtype),
    )(x, pos_table[:L])


# speedup vs baseline: 5.1093x; 1.0003x over previous
import jax
import jax.numpy as jnp
from jax.experimental import pallas as pl


_BLK_L = 2048


def _add_kernel(x_ref, pos_ref, o_ref):
    o_ref[...] = x_ref[...] + pos_ref[...][None]


def kernel(x, pos_table):
    B, L, D = x.shape
    blk_l = _BLK_L if L % _BLK_L == 0 else L
    grid = (L // blk_l, B)
    return pl.pallas_call(
        _add_kernel,
        grid=grid,
        in_specs=[
            pl.BlockSpec((1, blk_l, D), lambda l, b: (b, l, 0)),
            pl.BlockSpec((blk_l, D), lambda l, b: (l, 0)),
        ],
        out_specs=pl.BlockSpec((1, blk_l, D), lambda l, b: (b, l, 0)),
        out_shape=jax.ShapeDtypeStruct((B, L, D), x.dtype),
    )(x, pos_table[:L])


# speedup vs baseline: 5.2270x; 1.0230x over previous
import jax
import jax.numpy as jnp
from jax.experimental import pallas as pl
from jax.experimental.pallas import tpu as pltpu


_BLK_L = 2048


def _add_kernel(x_ref, pos_ref, o_ref):
    o_ref[...] = x_ref[...] + pos_ref[...][None]


def kernel(x, pos_table):
    B, L, D = x.shape
    blk_l = _BLK_L if L % _BLK_L == 0 else L
    grid = (L // blk_l, B // 2)
    return pl.pallas_call(
        _add_kernel,
        grid=grid,
        in_specs=[
            pl.BlockSpec((2, blk_l, D), lambda l, b: (b, l, 0)),
            pl.BlockSpec((blk_l, D), lambda l, b: (l, 0)),
        ],
        out_specs=pl.BlockSpec((2, blk_l, D), lambda l, b: (b, l, 0)),
        out_shape=jax.ShapeDtypeStruct((B, L, D), x.dtype),
        compiler_params=pltpu.CompilerParams(
            dimension_semantics=("parallel", "parallel"),
            vmem_limit_bytes=67000000,
        ),
    )(x, pos_table[:L])


# speedup vs baseline: 5.2317x; 1.0009x over previous
import jax
import jax.numpy as jnp
from jax.experimental import pallas as pl
from jax.experimental.pallas import tpu as pltpu


_BLK_L = 4096


def _add_kernel(x_ref, pos_ref, o_ref):
    o_ref[...] = x_ref[...] + pos_ref[...][None]


def kernel(x, pos_table):
    B, L, D = x.shape
    blk_l = _BLK_L if L % _BLK_L == 0 else L
    grid = (L // blk_l, B)
    return pl.pallas_call(
        _add_kernel,
        grid=grid,
        in_specs=[
            pl.BlockSpec((1, blk_l, D), lambda l, b: (b, l, 0)),
            pl.BlockSpec((blk_l, D), lambda l, b: (l, 0),
                         pipeline_mode=pl.Buffered(buffer_count=1)),
        ],
        out_specs=pl.BlockSpec((1, blk_l, D), lambda l, b: (b, l, 0)),
        out_shape=jax.ShapeDtypeStruct((B, L, D), x.dtype),
        compiler_params=pltpu.CompilerParams(
            dimension_semantics=("parallel", "parallel"),
            vmem_limit_bytes=67000000,
        ),
    )(x, pos_table[:L])
